# baseline (speedup 1.0000x reference)
import jax
import jax.numpy as jnp
from jax.experimental import pallas as pl
from jax.experimental.pallas import tpu as pltpu

_TN = 65536


def _mlp_kernel(x_ref, w1_ref, b1_ref, w2t_ref, b2_ref, w3t_ref, b3_ref,
                out_ref):
    x = x_ref[...].astype(jnp.bfloat16)
    b1 = b1_ref[...].T.astype(jnp.bfloat16)
    b2 = b2_ref[...].T.astype(jnp.bfloat16)
    h = jax.lax.dot_general(w1_ref[...].astype(jnp.bfloat16), x,
                            (((0,), (0,)), ((), ())),
                            preferred_element_type=jnp.float32)
    h = jnp.maximum(h.astype(jnp.bfloat16) + b1, 0.0)
    h = jnp.dot(w2t_ref[...].astype(jnp.bfloat16), h,
                preferred_element_type=jnp.float32)
    h = jnp.maximum(h.astype(jnp.bfloat16) + b2, 0.0)
    h = jnp.dot(w3t_ref[...].astype(jnp.bfloat16), h,
                preferred_element_type=jnp.float32)
    out_ref[...] = jnp.tanh(h + b3_ref[...].T).astype(out_ref.dtype)


def _round_up(x, m):
    return ((x + m - 1) // m) * m


@jax.jit
def _actor_forward(state, w1, b1, w2, b2, w3, b3):
    batch, in_dim = state.shape
    action_dim = w3.shape[1]

    xt = state.T
    w2t, w3t = w2.T, w3.T

    tn = min(_TN, _round_up(batch, 128))
    padded = _round_up(batch, tn)
    if padded != batch:
        xt = jnp.pad(xt, ((0, 0), (0, padded - batch)))

    grid = (padded // tn,)

    def resident(shape):
        return pl.BlockSpec(shape, lambda i, _s=shape: (0,) * len(_s))

    out_t = pl.pallas_call(
        _mlp_kernel,
        out_shape=jax.ShapeDtypeStruct((action_dim, padded), jnp.float32),
        grid=grid,
        in_specs=[
            pl.BlockSpec((in_dim, tn), lambda i: (0, i)),
            resident(w1.shape), resident(b1.shape),
            resident(w2t.shape), resident(b2.shape),
            resident(w3t.shape), resident(b3.shape),
        ],
        out_specs=pl.BlockSpec((action_dim, tn), lambda i: (0, i)),
        compiler_params=pltpu.CompilerParams(
            dimension_semantics=("parallel",),
            vmem_limit_bytes=64 * 1024 * 1024,
        ),
    )(xt, w1, b1, w2t, b2, w3t, b3)

    return out_t[:, :batch].T


def kernel(state, w1, b1, w2, b2, w3, b3):
    return _actor_forward(state, w1, b1, w2, b2, w3, b3)


# speedup vs baseline: 1.0328x; 1.0328x over previous
import jax
import jax.numpy as jnp
from jax.experimental import pallas as pl
from jax.experimental.pallas import tpu as pltpu

_TN = 65536


def _mlp_kernel(x_ref, w1_ref, b1_ref, w2t_ref, b2_ref, w3t_ref, b3_ref,
                out_ref):
    x = x_ref[...].astype(jnp.bfloat16)
    h = jax.lax.dot_general(w1_ref[...].astype(jnp.bfloat16), x,
                            (((0,), (0,)), ((), ())),
                            preferred_element_type=jnp.float32)
    h = jnp.maximum(h + b1_ref[...].T, 0.0).astype(jnp.bfloat16)
    h = jnp.dot(w2t_ref[...].astype(jnp.bfloat16), h,
                preferred_element_type=jnp.float32)
    h = jnp.maximum(h + b2_ref[...].T, 0.0).astype(jnp.bfloat16)
    h = jnp.dot(w3t_ref[...].astype(jnp.bfloat16), h,
                preferred_element_type=jnp.float32)
    out_ref[...] = jnp.tanh(h + b3_ref[...].T).astype(out_ref.dtype)


def _round_up(x, m):
    return ((x + m - 1) // m) * m


@jax.jit
def _actor_forward(state, w1, b1, w2, b2, w3, b3):
    batch, in_dim = state.shape
    action_dim = w3.shape[1]

    xt = state.T
    w2t, w3t = w2.T, w3.T

    tn = min(_TN, _round_up(batch, 128))
    padded = _round_up(batch, tn)
    if padded != batch:
        xt = jnp.pad(xt, ((0, 0), (0, padded - batch)))

    grid = (padded // tn,)

    def resident(shape):
        return pl.BlockSpec(shape, lambda i, _s=shape: (0,) * len(_s))

    out_t = pl.pallas_call(
        _mlp_kernel,
        out_shape=jax.ShapeDtypeStruct((action_dim, padded), jnp.float32),
        grid=grid,
        in_specs=[
            pl.BlockSpec((in_dim, tn), lambda i: (0, i)),
            resident(w1.shape), resident(b1.shape),
            resident(w2t.shape), resident(b2.shape),
            resident(w3t.shape), resident(b3.shape),
        ],
        out_specs=pl.BlockSpec((action_dim, tn), lambda i: (0, i)),
        compiler_params=pltpu.CompilerParams(
            dimension_semantics=("parallel",),
            vmem_limit_bytes=64 * 1024 * 1024,
        ),
    )(xt, w1, b1, w2t, b2, w3t, b3)

    return out_t[:, :batch].T


def kernel(state, w1, b1, w2, b2, w3, b3):
    return _actor_forward(state, w1, b1, w2, b2, w3, b3)
